# baseline (speedup 1.0000x reference)
import jax
import jax.numpy as jnp
from jax.experimental import pallas as pl
from jax.experimental.pallas import tpu as pltpu

_DECAY = 0.95
_CHUNK = 4096


def _l1_write_body(w_ref, k_ref, v_ref, m_ref, o_ref):
    c = pl.program_id(1)
    k = k_ref[...]
    v = v_ref[...]
    ones_row = jnp.ones((1, 128), dtype=jnp.float32)
    wb = jax.lax.dot_general(
        w_ref[...], ones_row, (((1,), (0,)), ((), ())),
        preferred_element_type=jnp.float32,
    )
    kw = k * wb
    p = jax.lax.dot_general(
        kw, v, (((0,), (0,)), ((), ())), preferred_element_type=jnp.float32
    )

    @pl.when(c == 0)
    def _():
        o_ref[...] = _DECAY * m_ref[...]

    o_ref[0, :, :] += p[:64, :64]
    o_ref[1, :, :] += p[64:, 64:]


def kernel(memory, keys, values, write_strengths):
    B, S, H, Dk = keys.shape
    Dv = values.shape[-1]
    BS = B * S
    n_pairs = H // 2
    n_chunks = BS // _CHUNK

    kp = keys.reshape(BS, H * Dk)
    vp = values.reshape(BS, H * Dv)
    w = write_strengths.reshape(BS, 1)

    out = pl.pallas_call(
        _l1_write_body,
        grid=(n_pairs, n_chunks),
        in_specs=[
            pl.BlockSpec((_CHUNK, 1), lambda g, c: (c, 0)),
            pl.BlockSpec((_CHUNK, 2 * Dk), lambda g, c: (c, g)),
            pl.BlockSpec((_CHUNK, 2 * Dv), lambda g, c: (c, g)),
            pl.BlockSpec((2, Dk, Dv), lambda g, c: (g, 0, 0)),
        ],
        out_specs=pl.BlockSpec((2, Dk, Dv), lambda g, c: (g, 0, 0)),
        out_shape=jax.ShapeDtypeStruct((H, Dk, Dv), jnp.float32),
        compiler_params=pltpu.CompilerParams(
            dimension_semantics=("parallel", "arbitrary"),
            vmem_limit_bytes=100 * 1024 * 1024,
        ),
    )(w, kp, vp, memory)
    return out


# speedup vs baseline: 1.0417x; 1.0417x over previous
import jax
import jax.numpy as jnp
from jax.experimental import pallas as pl
from jax.experimental.pallas import tpu as pltpu

_DECAY = 0.95
_CHUNK = 1024


def _l1_write_body(w_ref, k_ref, v_ref, m_ref, o_ref):
    c = pl.program_id(1)
    ones_row = jnp.ones((1, 128), dtype=jnp.float32)
    wb = jax.lax.dot_general(
        w_ref[...], ones_row, (((1,), (0,)), ((), ())),
        preferred_element_type=jnp.float32,
    )

    @pl.when(c == 0)
    def _():
        o_ref[0] = _DECAY * m_ref[0]

    n_pairs = k_ref.shape[1] // 128
    for p in range(n_pairs):
        sl = slice(128 * p, 128 * (p + 1))
        kw = k_ref[:, sl] * wb
        pp = jax.lax.dot_general(
            kw, v_ref[:, sl], (((0,), (0,)), ((), ())),
            preferred_element_type=jnp.float32,
        )
        o_ref[0, 2 * p, :, :] += pp[:64, :64]
        o_ref[0, 2 * p + 1, :, :] += pp[64:, 64:]


def kernel(memory, keys, values, write_strengths):
    B, S, H, Dk = keys.shape
    Dv = values.shape[-1]
    BS = B * S
    n_chunks_half = BS // 2 // _CHUNK

    kp = keys.reshape(BS, H * Dk)
    vp = values.reshape(BS, H * Dv)
    w = write_strengths.reshape(BS, 1)
    m2 = jnp.stack([memory, jnp.zeros_like(memory)])

    def row_idx(h, c):
        return (h * n_chunks_half + c, 0)

    partials = pl.pallas_call(
        _l1_write_body,
        grid=(2, n_chunks_half),
        in_specs=[
            pl.BlockSpec((_CHUNK, 1), row_idx),
            pl.BlockSpec((_CHUNK, H * Dk), row_idx),
            pl.BlockSpec((_CHUNK, H * Dv), row_idx),
            pl.BlockSpec((1, H, Dk, Dv), lambda h, c: (h, 0, 0, 0)),
        ],
        out_specs=pl.BlockSpec((1, H, Dk, Dv), lambda h, c: (h, 0, 0, 0)),
        out_shape=jax.ShapeDtypeStruct((2, H, Dk, Dv), jnp.float32),
        compiler_params=pltpu.CompilerParams(
            dimension_semantics=("parallel", "arbitrary"),
            vmem_limit_bytes=100 * 1024 * 1024,
        ),
    )(w, kp, vp, m2)
    return partials[0] + partials[1]


# speedup vs baseline: 5.5939x; 5.3702x over previous
import jax
import jax.numpy as jnp
from jax.experimental import pallas as pl
from jax.experimental.pallas import tpu as pltpu

_DECAY = 0.95


def _l1_write_body(w_ref, k_ref, v_ref, m_ref, o_ref):
    b = pl.program_id(1)
    k = k_ref[...]
    vw = v_ref[...] * w_ref[...]
    p = jax.lax.dot_general(
        k, vw, (((1,), (1,)), ((), ())), preferred_element_type=jnp.float32
    )

    @pl.when(b == 0)
    def _():
        o_ref[...] = _DECAY * m_ref[...]

    o_ref[...] += p


def kernel(memory, keys, values, write_strengths):
    B, S, H, Dk = keys.shape
    Dv = values.shape[-1]

    kt = jnp.transpose(keys, (0, 2, 3, 1))
    vt = jnp.transpose(values, (0, 2, 3, 1))
    w = write_strengths.reshape(1, B * S)

    return pl.pallas_call(
        _l1_write_body,
        grid=(H, B),
        in_specs=[
            pl.BlockSpec((1, S), lambda h, b: (0, b)),
            pl.BlockSpec((None, None, Dk, S), lambda h, b: (b, h, 0, 0)),
            pl.BlockSpec((None, None, Dv, S), lambda h, b: (b, h, 0, 0)),
            pl.BlockSpec((None, Dk, Dv), lambda h, b: (h, 0, 0)),
        ],
        out_specs=pl.BlockSpec((None, Dk, Dv), lambda h, b: (h, 0, 0)),
        out_shape=jax.ShapeDtypeStruct((H, Dk, Dv), jnp.float32),
        compiler_params=pltpu.CompilerParams(
            dimension_semantics=("parallel", "arbitrary"),
            vmem_limit_bytes=64 * 1024 * 1024,
        ),
    )(w, kt, vt, memory)
